# baseline (speedup 1.0000x reference)
import functools

import jax
import jax.numpy as jnp
from jax import lax
from jax.experimental import pallas as pl
from jax.experimental.pallas import tpu as pltpu
from jax.experimental.pallas import tpu_sc as plsc

B = 16384
F = 26
V = 100000
E = 16
H = 400
FE = F * E
EH = E // 2
FP = F * EH

NC = 2
NS = 16
NW = NC * NS
BF = B * F
PER_W = BF // NW
GRP = 128
GRP_PER_W = PER_W // GRP
GRP_PER_CHUNK = 13
CHUNK = GRP * GRP_PER_CHUNK
NCHUNK = GRP_PER_W // GRP_PER_CHUNK


def _sc_gather(table, idx3):
    mesh = plsc.VectorSubcoreMesh(
        core_axis_name="c", subcore_axis_name="s", num_cores=NC, num_subcores=NS
    )

    @functools.partial(
        pl.kernel,
        mesh=mesh,
        compiler_params=pltpu.CompilerParams(use_tc_tiling_on_sc=False),
        out_type=jax.ShapeDtypeStruct((BF, EH), jnp.int32),
        scratch_types=[
            pltpu.VMEM((GRP_PER_W, GRP), jnp.int32),
            pltpu.VMEM((CHUNK, EH), jnp.int32),
            pltpu.SemaphoreType.DMA,
        ],
    )
    def k(table_hbm, idx_hbm, out_hbm, idx_v, rows_v, sem):
        wid = lax.axis_index("s") * NC + lax.axis_index("c")
        base = wid * PER_W
        pltpu.sync_copy(idx_hbm.at[wid], idx_v)

        def chunk_body(c, carry):
            handles = []
            for j in range(GRP_PER_CHUNK):
                h = pltpu.async_copy(
                    table_hbm.at[idx_v.at[c * GRP_PER_CHUNK + j]],
                    rows_v.at[pl.ds(j * GRP, GRP)],
                    sem,
                )
                handles.append(h)
            for h in handles:
                h.wait()
            pltpu.sync_copy(rows_v, out_hbm.at[pl.ds(base + c * CHUNK, CHUNK)])
            return carry

        lax.fori_loop(0, NCHUNK, chunk_body, 0)

    return k(table, idx3)


FG = 2
GF = 13
GR = GF * E
GL = GF * EH
VC = 8192
NVC = -(-V // VC)


def _transpose_body(in_ref, out_ref):
    x = in_ref[...]
    u = lax.bitcast_convert_type(x, jnp.uint32)
    lsb = (u >> 16) & jnp.uint32(1)
    ub = ((u + jnp.uint32(0x7FFF) + lsb) >> 16).reshape(GF, 2, EH, VC)
    p = (ub[:, 0] | (ub[:, 1] << 16)).reshape(GL, VC)
    pz = jnp.concatenate(
        [p, jnp.zeros((128 - GL, VC), jnp.uint32)], axis=0)
    out_ref[...] = jnp.swapaxes(
        lax.bitcast_convert_type(pz, jnp.int32), 0, 1)[None]


def _tc_transpose(et):
    return pl.pallas_call(
        _transpose_body,
        grid=(FG, NVC),
        in_specs=[pl.BlockSpec((GR, VC), lambda g, c: (g, c))],
        out_specs=pl.BlockSpec((1, VC, 128), lambda g, c: (g, c, 0)),
        out_shape=jax.ShapeDtypeStruct((FG, V, 128), jnp.int32),
        compiler_params=pltpu.CompilerParams(
            dimension_semantics=("parallel", "parallel")),
    )(et)


BB = 1024


def _dense_body(xw_ref, pg_ref, pb_ref, w0a_lo_ref, w0a_hi_ref, w0b_ref,
                b0_ref, w1_ref, b1_ref, out_ref):
    w = lax.bitcast_convert_type(xw_ref[...], jnp.uint32)
    xlo = lax.bitcast_convert_type(w << 16, jnp.float32)
    xhi = lax.bitcast_convert_type(w & jnp.uint32(0xFFFF0000), jnp.float32)
    c_iota = lax.broadcasted_iota(jnp.int32, (FP, E), 0)
    e_iota = lax.broadcasted_iota(jnp.int32, (FP, E), 1)
    S_lo = (c_iota % EH == e_iota).astype(jnp.float32)
    S_hi = (c_iota % EH + EH == e_iota).astype(jnp.float32)
    z = (jnp.dot(xlo, S_lo, preferred_element_type=jnp.float32)
         + jnp.dot(xhi, S_hi, preferred_element_type=jnp.float32))
    i2 = lax.broadcasted_iota(jnp.int32, (E, E * E), 0)
    c2 = lax.broadcasted_iota(jnp.int32, (E, E * E), 1)
    A = (c2 // E == i2).astype(jnp.float32)
    Bm = (c2 % E == i2).astype(jnp.float32)
    rep = jnp.dot(z, A, preferred_element_type=jnp.float32)
    til = jnp.dot(z, Bm, preferred_element_type=jnp.float32)
    op = rep * til
    s1 = jnp.sum(z, axis=1, keepdims=True)
    s2 = jnp.sum(z * z, axis=1, keepdims=True)
    mean = s1 * s1 * (1.0 / (E * E))
    var = s2 * s2 * (1.0 / (E * E)) - mean * mean
    opn = (op - mean) * lax.rsqrt(var) * pg_ref[...] + pb_ref[...]
    h = (jnp.dot(xlo, w0a_lo_ref[...], preferred_element_type=jnp.float32)
         + jnp.dot(xhi, w0a_hi_ref[...], preferred_element_type=jnp.float32)
         + jnp.dot(opn, w0b_ref[...], preferred_element_type=jnp.float32)
         + b0_ref[...])
    h = jnp.maximum(h, 0.0)
    y = jnp.sum(h * w1_ref[...], axis=1, keepdims=True) + b1_ref[...]
    out_ref[...] = 1.0 / (1.0 + jnp.exp(-y))


def _dense(xw_flat, p_g, p_b, w0a_lo, w0a_hi, w0b, b0, w1, b1):
    grid = (B // BB,)
    return pl.pallas_call(
        _dense_body,
        grid=grid,
        in_specs=[
            pl.BlockSpec((BB, FP), lambda i: (i, 0)),
            pl.BlockSpec((1, E * E), lambda i: (0, 0)),
            pl.BlockSpec((1, E * E), lambda i: (0, 0)),
            pl.BlockSpec((FP, H), lambda i: (0, 0)),
            pl.BlockSpec((FP, H), lambda i: (0, 0)),
            pl.BlockSpec((E * E, H), lambda i: (0, 0)),
            pl.BlockSpec((1, H), lambda i: (0, 0)),
            pl.BlockSpec((1, H), lambda i: (0, 0)),
            pl.BlockSpec((1, 1), lambda i: (0, 0)),
        ],
        out_specs=pl.BlockSpec((BB, 1), lambda i: (i, 0)),
        out_shape=jax.ShapeDtypeStruct((B, 1), jnp.float32),
        compiler_params=pltpu.CompilerParams(
            dimension_semantics=("parallel",)),
    )(xw_flat, p_g, p_b, w0a_lo, w0a_hi, w0b, b0, w1, b1)


def kernel(indices, embed, p_g, p_b, w0, b0, w1, b1):
    f_ids = jnp.arange(F, dtype=jnp.int32)
    flat_idx = ((f_ids // GF) * (E * V))[None, :] \
        + indices.astype(jnp.int32) * E + (f_ids % GF)[None, :]
    idx3 = flat_idx.reshape(NW, GRP_PER_W, GRP)
    table = _tc_transpose(
        embed.transpose(0, 2, 1).reshape(FE, V)).reshape(FG * V * E, EH)
    xw = _sc_gather(table, idx3)
    xw_flat = xw.reshape(B, FP)
    w0a = w0[:FE].reshape(F, E, H)
    out = _dense(
        xw_flat,
        p_g.reshape(1, E * E),
        p_b.reshape(1, E * E),
        w0a[:, :EH].reshape(FP, H),
        w0a[:, EH:].reshape(FP, H),
        w0[FE:],
        b0.reshape(1, H),
        w1.reshape(1, H),
        b1.reshape(1, 1),
    )
    return out.reshape(B)


# speedup vs baseline: 1.0643x; 1.0643x over previous
import functools

import jax
import jax.numpy as jnp
from jax import lax
from jax.experimental import pallas as pl
from jax.experimental.pallas import tpu as pltpu
from jax.experimental.pallas import tpu_sc as plsc

B = 16384
F = 26
V = 100000
E = 16
H = 400
FE = F * E
EH = E // 2
FP = F * EH

NC = 2
NS = 16
NW = NC * NS
BF = B * F
BFH = BF // 2
PER_W = BFH // NW
GRP = 128
GRP_PER_W = PER_W // GRP
GRP_PER_CHUNK = 13
CHUNK = GRP * GRP_PER_CHUNK
NCHUNK = GRP_PER_W // GRP_PER_CHUNK


def _sc_gather(table, idx3):
    mesh = plsc.VectorSubcoreMesh(
        core_axis_name="c", subcore_axis_name="s", num_cores=NC, num_subcores=NS
    )

    @functools.partial(
        pl.kernel,
        mesh=mesh,
        compiler_params=pltpu.CompilerParams(use_tc_tiling_on_sc=False),
        out_type=jax.ShapeDtypeStruct((BFH, EH), jnp.int32),
        scratch_types=[
            pltpu.VMEM((GRP_PER_W, GRP), jnp.int32),
            pltpu.VMEM((CHUNK, EH), jnp.int32),
            pltpu.SemaphoreType.DMA,
        ],
    )
    def k(table_hbm, idx_hbm, out_hbm, idx_v, rows_v, sem):
        wid = lax.axis_index("s") * NC + lax.axis_index("c")
        base = wid * PER_W
        pltpu.sync_copy(idx_hbm.at[wid], idx_v)

        def chunk_body(c, carry):
            handles = []
            for j in range(GRP_PER_CHUNK):
                h = pltpu.async_copy(
                    table_hbm.at[idx_v.at[c * GRP_PER_CHUNK + j]],
                    rows_v.at[pl.ds(j * GRP, GRP)],
                    sem,
                )
                handles.append(h)
            for h in handles:
                h.wait()
            pltpu.sync_copy(rows_v, out_hbm.at[pl.ds(base + c * CHUNK, CHUNK)])
            return carry

        lax.fori_loop(0, NCHUNK, chunk_body, 0)

    return k(table, idx3)


FG = 2
GF = 13
GR = GF * E
GL = GF * EH
VC = 8192
NVC = -(-V // VC)


def _transpose_body(in_ref, out_ref):
    x = in_ref[...]
    u = lax.bitcast_convert_type(x, jnp.uint32)
    lsb = (u >> 16) & jnp.uint32(1)
    ub = ((u + jnp.uint32(0x7FFF) + lsb) >> 16).reshape(GF, 2, EH, VC)
    p = (ub[:, 0] | (ub[:, 1] << 16)).reshape(GL, VC)
    pz = jnp.concatenate(
        [p, jnp.zeros((128 - GL, VC), jnp.uint32)], axis=0)
    out_ref[...] = jnp.swapaxes(
        lax.bitcast_convert_type(pz, jnp.int32), 0, 1)[None]


def _tc_transpose(et):
    return pl.pallas_call(
        _transpose_body,
        grid=(FG, NVC),
        in_specs=[pl.BlockSpec((GR, VC), lambda g, c: (g, c))],
        out_specs=pl.BlockSpec((1, VC, 128), lambda g, c: (g, c, 0)),
        out_shape=jax.ShapeDtypeStruct((FG, V, 128), jnp.int32),
        compiler_params=pltpu.CompilerParams(
            dimension_semantics=("parallel", "parallel")),
    )(et)


BB = 1024


def _dense_body(xw_ref, pg_ref, pb_ref, w0a_lo_ref, w0a_hi_ref, w0b_ref,
                b0_ref, w1_ref, b1_ref, out_ref):
    w = lax.bitcast_convert_type(xw_ref[...], jnp.uint32)
    xlo = lax.bitcast_convert_type(w << 16, jnp.float32)
    xhi = lax.bitcast_convert_type(w & jnp.uint32(0xFFFF0000), jnp.float32)
    c_iota = lax.broadcasted_iota(jnp.int32, (FP, E), 0)
    e_iota = lax.broadcasted_iota(jnp.int32, (FP, E), 1)
    S_lo = (c_iota % EH == e_iota).astype(jnp.float32)
    S_hi = (c_iota % EH + EH == e_iota).astype(jnp.float32)
    z = (jnp.dot(xlo, S_lo, preferred_element_type=jnp.float32)
         + jnp.dot(xhi, S_hi, preferred_element_type=jnp.float32))
    i2 = lax.broadcasted_iota(jnp.int32, (E, E * E), 0)
    c2 = lax.broadcasted_iota(jnp.int32, (E, E * E), 1)
    A = (c2 // E == i2).astype(jnp.float32)
    Bm = (c2 % E == i2).astype(jnp.float32)
    rep = jnp.dot(z, A, preferred_element_type=jnp.float32)
    til = jnp.dot(z, Bm, preferred_element_type=jnp.float32)
    op = rep * til
    s1 = jnp.sum(z, axis=1, keepdims=True)
    s2 = jnp.sum(z * z, axis=1, keepdims=True)
    mean = s1 * s1 * (1.0 / (E * E))
    var = s2 * s2 * (1.0 / (E * E)) - mean * mean
    opn = (op - mean) * lax.rsqrt(var) * pg_ref[...] + pb_ref[...]
    h = (jnp.dot(xlo, w0a_lo_ref[...], preferred_element_type=jnp.float32)
         + jnp.dot(xhi, w0a_hi_ref[...], preferred_element_type=jnp.float32)
         + jnp.dot(opn, w0b_ref[...], preferred_element_type=jnp.float32)
         + b0_ref[...])
    h = jnp.maximum(h, 0.0)
    y = jnp.sum(h * w1_ref[...], axis=1, keepdims=True) + b1_ref[...]
    out_ref[...] = 1.0 / (1.0 + jnp.exp(-y))


BH = B // 2


def _dense(xw_flat, p_g, p_b, w0a_lo, w0a_hi, w0b, b0, w1, b1):
    grid = (BH // BB,)
    return pl.pallas_call(
        _dense_body,
        grid=grid,
        in_specs=[
            pl.BlockSpec((BB, FP), lambda i: (i, 0)),
            pl.BlockSpec((1, E * E), lambda i: (0, 0)),
            pl.BlockSpec((1, E * E), lambda i: (0, 0)),
            pl.BlockSpec((FP, H), lambda i: (0, 0)),
            pl.BlockSpec((FP, H), lambda i: (0, 0)),
            pl.BlockSpec((E * E, H), lambda i: (0, 0)),
            pl.BlockSpec((1, H), lambda i: (0, 0)),
            pl.BlockSpec((1, H), lambda i: (0, 0)),
            pl.BlockSpec((1, 1), lambda i: (0, 0)),
        ],
        out_specs=pl.BlockSpec((BB, 1), lambda i: (i, 0)),
        out_shape=jax.ShapeDtypeStruct((BH, 1), jnp.float32),
        compiler_params=pltpu.CompilerParams(
            dimension_semantics=("parallel",)),
    )(xw_flat, p_g, p_b, w0a_lo, w0a_hi, w0b, b0, w1, b1)


def kernel(indices, embed, p_g, p_b, w0, b0, w1, b1):
    f_ids = jnp.arange(F, dtype=jnp.int32)
    flat_idx = ((f_ids // GF) * (E * V))[None, :] \
        + indices.astype(jnp.int32) * E + (f_ids % GF)[None, :]
    idx_a = flat_idx[:BH].reshape(NW, GRP_PER_W, GRP)
    idx_b = flat_idx[BH:].reshape(NW, GRP_PER_W, GRP)
    table = _tc_transpose(
        embed.transpose(0, 2, 1).reshape(FE, V)).reshape(FG * V * E, EH)
    xwa = _sc_gather(table, idx_a).reshape(BH, FP)
    xwb = _sc_gather(table, idx_b).reshape(BH, FP)
    w0a = w0[:FE].reshape(F, E, H)
    dense_args = (
        p_g.reshape(1, E * E),
        p_b.reshape(1, E * E),
        w0a[:, :EH].reshape(FP, H),
        w0a[:, EH:].reshape(FP, H),
        w0[FE:],
        b0.reshape(1, H),
        w1.reshape(1, H),
        b1.reshape(1, 1),
    )
    out_a = _dense(xwa, *dense_args)
    out_b = _dense(xwb, *dense_args)
    return jnp.concatenate([out_a, out_b], axis=0).reshape(B)
